# Initial kernel scaffold; baseline (speedup 1.0000x reference)
import jax
import jax.numpy as jnp
from jax.experimental import pallas as pl


def kernel(x):
    raise NotImplementedError("write your pallas kernel here")


# speedup vs baseline: 11.4280x; 11.4280x over previous
import jax
import jax.numpy as jnp
from jax import lax
from jax.experimental import pallas as pl

K = 9
DILATION = 2
KK = K * DILATION - 1
TILE = 256
OUTW = 16


def _knn_kernel(xt_ref, xrow_ref, out_ref):
    xall = xt_ref[0]
    n = xall.shape[0]
    norm = jnp.sqrt(jnp.sum(xall * xall, axis=1, keepdims=True))
    xn = xall / jnp.maximum(norm, 1e-12)
    xr = xrow_ref[0]
    rnorm = jnp.sqrt(jnp.sum(xr * xr, axis=1, keepdims=True))
    rows = xr / jnp.maximum(rnorm, 1e-12)
    inner = -2.0 * lax.dot_general(
        rows, xn, (((1,), (1,)), ((), ())),
        preferred_element_type=jnp.float32)
    sq_rows = jnp.sum(rows * rows, axis=1, keepdims=True)
    sq_all = jnp.sum(xn * xn, axis=1)[None, :]
    dist = (sq_rows + inner) + sq_all
    iota = lax.broadcasted_iota(jnp.int32, (TILE, n), 1)
    cols = []
    d = dist
    for t in range(KK):
        v = jnp.min(d, axis=1, keepdims=True)
        idx = jnp.min(jnp.where(d == v, iota, n), axis=1)
        if t % DILATION == 0:
            cols.append(idx)
        if t < KK - 1:
            d = jnp.where(iota == idx[:, None], jnp.float32(jnp.inf), d)
    out = jnp.stack(cols, axis=1)
    out_ref[0] = jnp.pad(out, ((0, 0), (0, OUTW - K)))


def kernel(x):
    b, c, n, _ = x.shape
    xt = jnp.transpose(x[..., 0], (0, 2, 1))
    nn = pl.pallas_call(
        _knn_kernel,
        grid=(b, n // TILE),
        in_specs=[pl.BlockSpec((1, n, c), lambda bb, ii: (bb, 0, 0)),
                  pl.BlockSpec((1, TILE, c), lambda bb, ii: (bb, ii, 0))],
        out_specs=pl.BlockSpec((1, TILE, OUTW), lambda bb, ii: (bb, ii, 0)),
        out_shape=jax.ShapeDtypeStruct((b, n, OUTW), jnp.int32),
    )(xt, xt)
    nn9 = nn[..., :K]
    center = jnp.broadcast_to(
        jnp.arange(n, dtype=jnp.int32)[None, :, None], (b, n, K))
    return jnp.stack((nn9, center), axis=0)


# speedup vs baseline: 14.6003x; 1.2776x over previous
import jax
import jax.numpy as jnp
from jax import lax
from jax.experimental import pallas as pl

K = 9
DILATION = 2
KK = K * DILATION - 1
TILE = 512
OUTW = 16


def _knn_kernel(xt_ref, xrow_ref, out_ref):
    xall = xt_ref[0]
    n = xall.shape[0]
    norm = jnp.sqrt(jnp.sum(xall * xall, axis=1, keepdims=True))
    xn = xall / jnp.maximum(norm, 1e-12)
    xr = xrow_ref[0]
    rnorm = jnp.sqrt(jnp.sum(xr * xr, axis=1, keepdims=True))
    rows = xr / jnp.maximum(rnorm, 1e-12)
    inner = -2.0 * lax.dot_general(
        rows, xn, (((1,), (1,)), ((), ())),
        preferred_element_type=jnp.float32)
    sq_rows = jnp.sum(rows * rows, axis=1, keepdims=True)
    sq_all = jnp.sum(xn * xn, axis=1)[None, :]
    dist = (sq_rows + inner) + sq_all
    fiota = lax.broadcasted_iota(jnp.int32, (TILE, n), 1).astype(jnp.float32)
    big = jnp.float32(n)
    cols = []
    d = dist
    for t in range(KK):
        v = jnp.min(d, axis=1, keepdims=True)
        fidx = jnp.min(jnp.where(d == v, fiota, big), axis=1)
        if t % DILATION == 0:
            cols.append(fidx)
        if t < KK - 1:
            d = jnp.where(fiota == fidx[:, None], jnp.float32(jnp.inf), d)
    out = jnp.stack(cols, axis=1).astype(jnp.int32)
    out_ref[0] = jnp.pad(out, ((0, 0), (0, OUTW - K)))


def kernel(x):
    b, c, n, _ = x.shape
    xt = jnp.transpose(x[..., 0], (0, 2, 1))
    nn = pl.pallas_call(
        _knn_kernel,
        grid=(b, n // TILE),
        in_specs=[pl.BlockSpec((1, n, c), lambda bb, ii: (bb, 0, 0)),
                  pl.BlockSpec((1, TILE, c), lambda bb, ii: (bb, ii, 0))],
        out_specs=pl.BlockSpec((1, TILE, OUTW), lambda bb, ii: (bb, ii, 0)),
        out_shape=jax.ShapeDtypeStruct((b, n, OUTW), jnp.int32),
    )(xt, xt)
    nn9 = nn[..., :K]
    center = jnp.broadcast_to(
        jnp.arange(n, dtype=jnp.int32)[None, :, None], (b, n, K))
    return jnp.stack((nn9, center), axis=0)
